# baseline (speedup 1.0000x reference)
import functools

import jax
import jax.numpy as jnp
from jax import lax
from jax.experimental import pallas as pl
from jax.experimental.pallas import tpu as pltpu
from jax.experimental.pallas import tpu_sc as plsc

N_A = 320000
N_B = 320000
D = 128
NUM_SEG = 40000
NJ = D // 16

NC = 2
NS = 16
NW = NC * NS

SEG_CHUNK = 200
NUM_CHUNKS = NUM_SEG // SEG_CHUNK
CHUNK_ITERS = -(-NUM_CHUNKS // NW)
RB = 256
RB_PAD = RB + 8
IDS_PAD = 296
STARTS_PAD = 224
U = 8

BGC = 80
BPW = N_B // NW
GPW = BPW // BGC
NBUF = 5
NGROUP = GPW // NBUF

_mesh = plsc.VectorSubcoreMesh(core_axis_name="c", subcore_axis_name="s")


@functools.partial(
    pl.kernel,
    out_type=jax.ShapeDtypeStruct((NUM_SEG * D,), jnp.float32),
    mesh=_mesh,
    scratch_types=[
        pltpu.VMEM((RB_PAD * D,), jnp.float32),
        pltpu.VMEM((RB_PAD * D,), jnp.float32),
        pltpu.VMEM((IDS_PAD,), jnp.int32),
        pltpu.VMEM((IDS_PAD,), jnp.int32),
        pltpu.VMEM((STARTS_PAD,), jnp.int32),
        pltpu.VMEM((SEG_CHUNK * D,), jnp.float32),
        pltpu.SemaphoreType.DMA,
        pltpu.SemaphoreType.DMA,
        pltpu.SemaphoreType.DMA,
        pltpu.SemaphoreType.DMA,
    ],
)
def _segmax(a_hbm, ids_hbm, starts_hbm, pooled_hbm,
            row_buf0, row_buf1, ids_vmem0, ids_vmem1, starts_vmem, out_buf,
            isem0, isem1, rsem0, rsem1):
    wid = lax.axis_index("s") * NC + lax.axis_index("c")
    pltpu.sync_copy(starts_hbm, starts_vmem)

    zero16 = jnp.zeros((16,), jnp.float32)
    bufs = ((row_buf0, ids_vmem0, isem0, rsem0),
            (row_buf1, ids_vmem1, isem1, rsem1))

    def start_block(b, lo, rb, iv, isem, rsem):
        base = lo + b * RB
        win_i = jnp.minimum((base // 8) * 8, N_A)
        win_r = jnp.minimum(win_i, N_A - RB_PAD)
        pltpu.async_copy(ids_hbm.at[pl.ds(win_i, IDS_PAD)], iv, isem)
        pltpu.async_copy(a_hbm.at[pl.ds(win_r * D, RB_PAD * D)], rb, rsem)

    def drain_block(rb, iv, isem, rsem):
        pltpu.make_async_copy(ids_hbm.at[pl.ds(0, IDS_PAD)], iv, isem).wait()
        pltpu.make_async_copy(a_hbm.at[pl.ds(0, RB_PAD * D)], rb, rsem).wait()

    for t in range(CHUNK_ITERS):
        c = wid + t * NW

        @pl.when(c < NUM_CHUNKS)
        def _(c=c):
            sv = starts_vmem[pl.ds(c, 16)]
            lo = sv[0]
            hi = sv[1]
            c0 = c * SEG_CHUNK

            @plsc.parallel_loop(0, SEG_CHUNK * NJ, unroll=8)
            def _zero(s):
                out_buf[pl.ds(s * 16, 16)] = zero16

            nrows = hi - lo
            nblocks = (nrows + RB - 1) // RB
            npairs = (jnp.maximum(nblocks, 1) + 1) // 2

            start_block(0, lo, *bufs[0])

            def row_loop(carry, b, rb, iv, lo, hi, c0):
                base = lo + b * RB
                n = jnp.clip(hi - base, 0, RB)
                win_i = jnp.minimum((base // 8) * 8, N_A)
                win_r = jnp.minimum(win_i, N_A - RB_PAD)
                skew_i = base - win_i
                skew_r = base - win_r
                ninf_i = lax.bitcast_convert_type(
                    jnp.full((16,), -jnp.inf, jnp.float32), jnp.int32)
                bidx = jnp.zeros((16, 1), jnp.int32)

                @plsc.parallel_loop(0, n, carry=carry, unroll=8)
                def rc(i, acc):
                    p = skew_i + i + 8
                    idv = iv[pl.ds(p, 16)]
                    idvp = iv[pl.ds(p - 1, 16)]
                    d = idv ^ idvp
                    nz = lax.shift_right_arithmetic(d | (0 - d), 31)
                    nm = lax.gather(
                        nz, bidx,
                        lax.GatherDimensionNumbers(
                            offset_dims=(), collapsed_slice_dims=(0,),
                            start_index_map=(0,)),
                        slice_sizes=(1,),
                        mode=lax.GatherScatterMode.PROMISE_IN_BOUNDS)
                    sm = nm ^ jnp.int32(-1)
                    sid = idv[0]
                    nxt = idv[1]
                    nacc = []
                    for j in range(NJ):
                        v = rb[pl.ds((skew_r + i) * D + j * 16, 16)]
                        acc_i = lax.bitcast_convert_type(acc[j], jnp.int32)
                        masked = lax.bitcast_convert_type(
                            (acc_i & sm) | (ninf_i & nm), jnp.float32)
                        nacc.append(jnp.maximum(masked, v))

                    @pl.when(sid != nxt)
                    def _():
                        for j in range(NJ):
                            out_buf[pl.ds((sid - c0) * D + j * 16, 16)] = nacc[j]

                    return tuple(nacc)

                return rc

            def outer_body(g, carry, lo=lo, hi=hi, c0=c0):
                for u in range(2):
                    b = 2 * g + u
                    rb, iv, isem, rsem = bufs[u]
                    orb, oiv, oisem, orsem = bufs[1 - u]
                    drain_block(rb, iv, isem, rsem)

                    @pl.when(b + 1 < 2 * npairs)
                    def _(b=b, lo=lo, orb=orb, oiv=oiv, oisem=oisem,
                          orsem=orsem):
                        start_block(b + 1, lo, orb, oiv, oisem, orsem)

                    carry = row_loop(carry, b, rb, iv, lo, hi, c0)
                return carry

            init = tuple(
                jnp.full((16,), -jnp.inf, jnp.float32) for _ in range(NJ))
            lax.fori_loop(0, npairs, outer_body, init)

            pltpu.sync_copy(out_buf,
                            pooled_hbm.at[pl.ds(c0 * D, SEG_CHUNK * D)])


@functools.partial(
    pl.kernel,
    out_type=jax.ShapeDtypeStruct((N_B, D), jnp.float32),
    mesh=_mesh,
    scratch_types=[
        pltpu.VMEM((BPW,), jnp.int32),
        pltpu.VMEM((NBUF, BGC, D), jnp.float32),
        pltpu.SemaphoreType.DMA,
        pltpu.SemaphoreType.DMA,
    ],
)
def _unpool(pooled_hbm, bids_hbm, out_hbm, idx_all, rows, gsem, wsem):
    wid = lax.axis_index("s") * NC + lax.axis_index("c")
    pltpu.sync_copy(bids_hbm.at[pl.ds(wid * BPW, BPW)], idx_all)

    def group_body(g, carry):
        ghandles = []
        for b in range(NBUF):
            k = g * NBUF + b
            ghandles.append(
                pltpu.async_copy(pooled_hbm.at[idx_all.at[pl.ds(k * BGC, BGC)]],
                                 rows.at[b], gsem))
        whandles = []
        for b in range(NBUF):
            k = g * NBUF + b
            ghandles[b].wait()
            off = wid * BPW + k * BGC
            whandles.append(
                pltpu.async_copy(rows.at[b], out_hbm.at[pl.ds(off, BGC)], wsem))
        for wh in whandles:
            wh.wait()
        return carry

    lax.fori_loop(0, NGROUP, group_body, 0)


def kernel(a_feat, a_seg_ids, b_seg_ids, num_segments):
    del num_segments
    bounds = jnp.arange(0, NUM_SEG + 1, SEG_CHUNK, dtype=jnp.int32)
    starts = jnp.searchsorted(a_seg_ids, bounds, side="left").astype(jnp.int32)
    starts = jnp.concatenate(
        [starts, jnp.full((STARTS_PAD - NUM_CHUNKS - 1,), N_A, jnp.int32)])
    ids_padded = jnp.concatenate(
        [jnp.full((8,), -1, jnp.int32), a_seg_ids,
         jnp.full((IDS_PAD,), NUM_SEG, jnp.int32)])
    pooled = _segmax(a_feat.reshape(N_A * D), ids_padded, starts)
    return _unpool(pooled.reshape(NUM_SEG, D), b_seg_ids)


# speedup vs baseline: 1.2567x; 1.2567x over previous
import functools

import jax
import jax.numpy as jnp
from jax import lax
from jax.experimental import pallas as pl
from jax.experimental.pallas import tpu as pltpu
from jax.experimental.pallas import tpu_sc as plsc

N_A = 320000
N_B = 320000
D = 128
NUM_SEG = 40000
NJ = D // 16

NC = 2
NS = 16
NW = NC * NS

SEG_CHUNK = 200
NUM_CHUNKS = NUM_SEG // SEG_CHUNK
CHUNK_ITERS = -(-NUM_CHUNKS // NW)
RB = 320
RB_PAD = RB + 8
IDS_PAD = 344
STARTS_PAD = 224
U = 8

BGC = 80
BPW = N_B // NW
GPW = BPW // BGC
NBUF = 5
NGROUP = GPW // NBUF

_mesh = plsc.VectorSubcoreMesh(core_axis_name="c", subcore_axis_name="s")


@functools.partial(
    pl.kernel,
    out_type=jax.ShapeDtypeStruct((NUM_SEG * D,), jnp.float32),
    mesh=_mesh,
    scratch_types=[
        pltpu.VMEM((RB_PAD * D,), jnp.float32),
        pltpu.VMEM((RB_PAD * D,), jnp.float32),
        pltpu.VMEM((IDS_PAD,), jnp.int32),
        pltpu.VMEM((IDS_PAD,), jnp.int32),
        pltpu.VMEM((STARTS_PAD,), jnp.int32),
        pltpu.VMEM((SEG_CHUNK * D,), jnp.float32),
        pltpu.SemaphoreType.DMA,
        pltpu.SemaphoreType.DMA,
        pltpu.SemaphoreType.DMA,
        pltpu.SemaphoreType.DMA,
    ],
)
def _segmax(a_hbm, ids_hbm, starts_hbm, pooled_hbm,
            row_buf0, row_buf1, ids_vmem0, ids_vmem1, starts_vmem, out_buf,
            isem0, isem1, rsem0, rsem1):
    wid = lax.axis_index("s") * NC + lax.axis_index("c")
    pltpu.sync_copy(starts_hbm, starts_vmem)

    zero16 = jnp.zeros((16,), jnp.float32)
    bufs = ((row_buf0, ids_vmem0, isem0, rsem0),
            (row_buf1, ids_vmem1, isem1, rsem1))

    def start_block(b, lo, rb, iv, isem, rsem):
        base = lo + b * RB
        win_i = jnp.minimum((base // 8) * 8, N_A)
        win_r = jnp.minimum(win_i, N_A - RB_PAD)
        pltpu.async_copy(ids_hbm.at[pl.ds(win_i, IDS_PAD)], iv, isem)
        pltpu.async_copy(a_hbm.at[pl.ds(win_r * D, RB_PAD * D)], rb, rsem)

    def drain_block(rb, iv, isem, rsem):
        pltpu.make_async_copy(ids_hbm.at[pl.ds(0, IDS_PAD)], iv, isem).wait()
        pltpu.make_async_copy(a_hbm.at[pl.ds(0, RB_PAD * D)], rb, rsem).wait()

    for t in range(CHUNK_ITERS):
        c = wid + t * NW

        @pl.when(c < NUM_CHUNKS)
        def _(c=c):
            sv = starts_vmem[pl.ds(c, 16)]
            lo = sv[0]
            hi = sv[1]
            c0 = c * SEG_CHUNK

            @plsc.parallel_loop(0, SEG_CHUNK * NJ, unroll=8)
            def _zero(s):
                out_buf[pl.ds(s * 16, 16)] = zero16

            nrows = hi - lo
            nblocks = (nrows + RB - 1) // RB
            npairs = (jnp.maximum(nblocks, 1) + 1) // 2

            start_block(0, lo, *bufs[0])

            def row_loop(carry, b, rb, iv, lo, hi, c0):
                base = lo + b * RB
                n = jnp.clip(hi - base, 0, RB)
                win_i = jnp.minimum((base // 8) * 8, N_A)
                win_r = jnp.minimum(win_i, N_A - RB_PAD)
                skew_i = base - win_i
                skew_r = base - win_r

                @plsc.parallel_loop(0, n, carry=carry, unroll=8)
                def rc(i, rc):
                    pid = rc[0]
                    acc = rc[1:]
                    sid = iv[pl.ds(skew_i + i, 16)][0]
                    is_new = sid != pid

                    @pl.when(is_new & (pid >= 0))
                    def _():
                        for j in range(NJ):
                            out_buf[pl.ds((pid - c0) * D + j * 16, 16)] = acc[j]

                    nacc = []
                    for j in range(NJ):
                        v = rb[pl.ds((skew_r + i) * D + j * 16, 16)]
                        nacc.append(jnp.where(is_new, v, jnp.maximum(acc[j], v)))
                    return (sid,) + tuple(nacc)

                return rc

            def outer_body(g, carry, lo=lo, hi=hi, c0=c0):
                for u in range(2):
                    b = 2 * g + u
                    rb, iv, isem, rsem = bufs[u]
                    orb, oiv, oisem, orsem = bufs[1 - u]
                    drain_block(rb, iv, isem, rsem)

                    @pl.when(b + 1 < 2 * npairs)
                    def _(b=b, lo=lo, orb=orb, oiv=oiv, oisem=oisem,
                          orsem=orsem):
                        start_block(b + 1, lo, orb, oiv, oisem, orsem)

                    carry = row_loop(carry, b, rb, iv, lo, hi, c0)
                return carry

            init = (jnp.int32(-1),) + tuple(
                jnp.full((16,), -jnp.inf, jnp.float32) for _ in range(NJ))
            fin = lax.fori_loop(0, npairs, outer_body, init)
            last_id = fin[0]
            last_acc = fin[1:]

            @pl.when(last_id >= 0)
            def _(last_id=last_id, last_acc=last_acc, c0=c0):
                for j in range(NJ):
                    out_buf[pl.ds((last_id - c0) * D + j * 16, 16)] = last_acc[j]

            pltpu.sync_copy(out_buf,
                            pooled_hbm.at[pl.ds(c0 * D, SEG_CHUNK * D)])


@functools.partial(
    pl.kernel,
    out_type=jax.ShapeDtypeStruct((N_B, D), jnp.float32),
    mesh=_mesh,
    scratch_types=[
        pltpu.VMEM((BPW,), jnp.int32),
        pltpu.VMEM((NBUF, BGC, D), jnp.float32),
        pltpu.SemaphoreType.DMA,
        pltpu.SemaphoreType.DMA,
    ],
)
def _unpool(pooled_hbm, bids_hbm, out_hbm, idx_all, rows, gsem, wsem):
    wid = lax.axis_index("s") * NC + lax.axis_index("c")
    pltpu.sync_copy(bids_hbm.at[pl.ds(wid * BPW, BPW)], idx_all)

    def group_body(g, carry):
        ghandles = []
        for b in range(NBUF):
            k = g * NBUF + b
            ghandles.append(
                pltpu.async_copy(pooled_hbm.at[idx_all.at[pl.ds(k * BGC, BGC)]],
                                 rows.at[b], gsem))
        whandles = []
        for b in range(NBUF):
            k = g * NBUF + b
            ghandles[b].wait()
            off = wid * BPW + k * BGC
            whandles.append(
                pltpu.async_copy(rows.at[b], out_hbm.at[pl.ds(off, BGC)], wsem))
        for wh in whandles:
            wh.wait()
        return carry

    lax.fori_loop(0, NGROUP, group_body, 0)


def kernel(a_feat, a_seg_ids, b_seg_ids, num_segments):
    del num_segments
    bounds = jnp.arange(0, NUM_SEG + 1, SEG_CHUNK, dtype=jnp.int32)
    starts = jnp.searchsorted(a_seg_ids, bounds, side="left").astype(jnp.int32)
    starts = jnp.concatenate(
        [starts, jnp.full((STARTS_PAD - NUM_CHUNKS - 1,), N_A, jnp.int32)])
    ids_padded = jnp.concatenate(
        [a_seg_ids, jnp.full((IDS_PAD,), NUM_SEG, jnp.int32)])
    pooled = _segmax(a_feat.reshape(N_A * D), ids_padded, starts)
    return _unpool(pooled.reshape(NUM_SEG, D), b_seg_ids)
